# Initial kernel scaffold; baseline (speedup 1.0000x reference)
import jax
import jax.numpy as jnp
from jax.experimental import pallas as pl


def kernel(x, edge_index, W1, b1, W2, b2):
    raise NotImplementedError("write your pallas kernel here")


# speedup vs baseline: 21.0158x; 21.0158x over previous
import functools

import jax
import jax.numpy as jnp
from jax import lax
from jax.experimental import pallas as pl
from jax.experimental.pallas import tpu as pltpu
from jax.experimental.pallas import tpu_sc as plsc

N = 10000
E = 320000
DP = 32
WD = 16
NC = 2
NS = 16
NW = NC * NS
K = 128
NCHUNK = 80
EPAD = NW * NCHUNK * K
NPAD = 10240
RPS = NPAD // NS

def _sc_deg_body(dst_hbm, zeros_hbm, ones_hbm, out_hbm, dst_v, ones_v, acc):
    c = lax.axis_index("c")
    s = lax.axis_index("s")
    wid = s * NC + c
    pltpu.sync_copy(dst_hbm.at[wid], dst_v)
    pltpu.sync_copy(ones_hbm, ones_v)
    pltpu.sync_copy(zeros_hbm.at[pl.ds(s * RPS, RPS)], acc.at[pl.ds(s * RPS, RPS)])
    plsc.subcore_barrier()

    def step(j, carry):
        pltpu.sync_copy(ones_v, acc.at[dst_v.at[j]], add=True)
        return carry

    lax.fori_loop(0, NCHUNK, step, 0)
    plsc.subcore_barrier()
    pltpu.sync_copy(acc.at[pl.ds(s * RPS, RPS)],
                    out_hbm.at[pl.ds(c * NPAD + s * RPS, RPS)])


def _sc_agg_body(g_hbm, src_hbm, dst_hbm, zeros_hbm, out_hbm,
                 src_v, dst_v, rows_v, acc, sem):
    c = lax.axis_index("c")
    s = lax.axis_index("s")
    wid = s * NC + c
    pltpu.sync_copy(src_hbm.at[wid], src_v)
    pltpu.sync_copy(dst_hbm.at[wid], dst_v)
    pltpu.sync_copy(zeros_hbm.at[pl.ds(s * RPS, RPS)], acc.at[pl.ds(s * RPS, RPS)])
    plsc.subcore_barrier()

    def step(j, carry):
        pltpu.async_copy(g_hbm.at[src_v.at[j]], rows_v, sem).wait()
        pltpu.sync_copy(rows_v, acc.at[dst_v.at[j]], add=True)
        return carry

    lax.fori_loop(0, NCHUNK, step, 0)
    plsc.subcore_barrier()
    pltpu.sync_copy(acc.at[pl.ds(s * RPS, RPS)],
                    out_hbm.at[pl.ds(c * NPAD + s * RPS, RPS)])


@functools.cache
def _sc_kernels():
    mesh = plsc.VectorSubcoreMesh(core_axis_name="c", subcore_axis_name="s")
    params = pltpu.CompilerParams(use_tc_tiling_on_sc=False)
    sc_deg = pl.kernel(
        _sc_deg_body,
        out_type=jax.ShapeDtypeStruct((NC * NPAD, WD), jnp.float32),
        mesh=mesh,
        compiler_params=params,
        scratch_types=[
            pltpu.VMEM((NCHUNK, K), jnp.int32),
            pltpu.VMEM((K, WD), jnp.float32),
            pltpu.VMEM_SHARED((NPAD, WD), jnp.float32),
        ],
    )
    sc_agg = pl.kernel(
        _sc_agg_body,
        out_type=jax.ShapeDtypeStruct((NC * NPAD, DP), jnp.float32),
        mesh=mesh,
        compiler_params=params,
        scratch_types=[
            pltpu.VMEM((NCHUNK, K), jnp.int32),
            pltpu.VMEM((NCHUNK, K), jnp.int32),
            pltpu.VMEM((K, DP), jnp.float32),
            pltpu.VMEM_SHARED((NPAD, DP), jnp.float32),
            pltpu.SemaphoreType.DMA,
        ],
    )
    return sc_deg, sc_agg


_NB = 10
_BR = N // _NB


def _dinv_of(deg_ref):
    d = deg_ref[0, :, 0:1] + deg_ref[1, :, 0:1] + 1.0
    return lax.rsqrt(d)


def _tc1_body(deg_ref, x_ref, w_ref, o_ref):
    dinv = _dinv_of(deg_ref)
    h = jnp.dot(x_ref[...], w_ref[...], preferred_element_type=jnp.float32)
    o_ref[...] = h * dinv


def _tc2_body(deg_ref, a_ref, g_ref, w_ref, b_ref, o_ref):
    dinv = _dinv_of(deg_ref)
    z = jnp.maximum((a_ref[0] + a_ref[1] + g_ref[...]) * dinv + b_ref[...], 0.0)
    o_ref[...] = jnp.dot(z, w_ref[...], preferred_element_type=jnp.float32) * dinv


def _tc3_body(deg_ref, a_ref, g_ref, b_ref, o_ref):
    dinv = _dinv_of(deg_ref)
    o_ref[...] = (a_ref[0] + a_ref[1] + g_ref[...]) * dinv + b_ref[...]


_deg_spec = pl.BlockSpec((2, _BR, WD), lambda i: (0, i, 0))
_agg_spec = pl.BlockSpec((2, _BR, DP), lambda i: (0, i, 0))
_row_spec = pl.BlockSpec((_BR, DP), lambda i: (i, 0))
_out_spec = pl.BlockSpec((_BR, DP), lambda i: (i, 0))


def _tc1(degp, x, w1p):
    return pl.pallas_call(
        _tc1_body,
        grid=(_NB,),
        in_specs=[_deg_spec,
                  pl.BlockSpec((_BR, 128), lambda i: (i, 0)),
                  pl.BlockSpec((128, DP), lambda i: (0, 0))],
        out_specs=_out_spec,
        out_shape=jax.ShapeDtypeStruct((N, DP), jnp.float32),
    )(degp, x, w1p)


def _tc2(degp, aggp, g1, w2p, b1p):
    return pl.pallas_call(
        _tc2_body,
        grid=(_NB,),
        in_specs=[_deg_spec, _agg_spec, _row_spec,
                  pl.BlockSpec((DP, DP), lambda i: (0, 0)),
                  pl.BlockSpec((1, DP), lambda i: (0, 0))],
        out_specs=_out_spec,
        out_shape=jax.ShapeDtypeStruct((N, DP), jnp.float32),
    )(degp, aggp, g1, w2p, b1p)


def _tc3(degp, aggp, g2, b2p):
    return pl.pallas_call(
        _tc3_body,
        grid=(_NB,),
        in_specs=[_deg_spec, _agg_spec, _row_spec,
                  pl.BlockSpec((1, DP), lambda i: (0, 0))],
        out_specs=_out_spec,
        out_shape=jax.ShapeDtypeStruct((N, DP), jnp.float32),
    )(degp, aggp, g2, b2p)


def kernel(x, edge_index, W1, b1, W2, b2):
    ei = edge_index.astype(jnp.int32)
    src = jnp.concatenate([ei[0], jnp.zeros((EPAD - E,), jnp.int32)])
    dst = jnp.concatenate([ei[1], jnp.full((EPAD - E,), N, jnp.int32)])
    srcI = src.reshape(NW, NCHUNK, K)
    dstI = dst.reshape(NW, NCHUNK, K)

    w1p = jnp.zeros((128, DP), jnp.float32).at[:, :30].set(W1)
    b1p = jnp.zeros((1, DP), jnp.float32).at[0, :30].set(b1)
    w2p = jnp.zeros((DP, DP), jnp.float32).at[:30, :25].set(W2)
    b2p = jnp.zeros((1, DP), jnp.float32).at[0, :25].set(b2)

    z16 = jnp.zeros((NPAD, WD), jnp.float32)
    z32 = jnp.zeros((NPAD, DP), jnp.float32)
    ones16 = jnp.ones((K, WD), jnp.float32)

    sc_deg, sc_agg = _sc_kernels()
    degp = sc_deg(dstI, z16, ones16).reshape(NC, NPAD, WD)[:, :N, :]
    g1 = _tc1(degp, x, w1p)
    a1 = sc_agg(g1, srcI, dstI, z32).reshape(NC, NPAD, DP)[:, :N, :]
    g2 = _tc2(degp, a1, g1, w2p, b1p)
    a2 = sc_agg(g2, srcI, dstI, z32).reshape(NC, NPAD, DP)[:, :N, :]
    out = _tc3(degp, a2, g2, b2p)
    return out[:, :25]


# speedup vs baseline: 25.6062x; 1.2184x over previous
import functools

import jax
import jax.numpy as jnp
from jax import lax
from jax.experimental import pallas as pl
from jax.experimental.pallas import tpu as pltpu
from jax.experimental.pallas import tpu_sc as plsc

N = 10000
E = 320000
DP = 32
WD = 16
NC = 2
NS = 16
NW = NC * NS
K = 128
NCHUNK = 80
EPAD = NW * NCHUNK * K
NPAD = 10240
RPS = NPAD // NS

def _sc_deg_body(dst_hbm, zeros_hbm, ones_hbm, out_hbm, dst_v, ones_v, acc):
    c = lax.axis_index("c")
    s = lax.axis_index("s")
    wid = s * NC + c
    pltpu.sync_copy(dst_hbm.at[wid], dst_v)
    pltpu.sync_copy(ones_hbm, ones_v)
    pltpu.sync_copy(zeros_hbm.at[pl.ds(s * RPS, RPS)], acc.at[pl.ds(s * RPS, RPS)])
    plsc.subcore_barrier()

    def step(j, carry):
        pltpu.sync_copy(ones_v, acc.at[dst_v.at[j]], add=True)
        return carry

    lax.fori_loop(0, NCHUNK, step, 0)
    plsc.subcore_barrier()
    pltpu.sync_copy(acc.at[pl.ds(s * RPS, RPS)],
                    out_hbm.at[pl.ds(c * NPAD + s * RPS, RPS)])


def _sc_agg_body(g_hbm, src_hbm, dst_hbm, zeros_hbm, out_hbm,
                 src_v, dst_v, rows_a, rows_b, acc, sem_a, sem_b):
    c = lax.axis_index("c")
    s = lax.axis_index("s")
    wid = s * NC + c
    pltpu.sync_copy(src_hbm.at[wid], src_v)
    pltpu.sync_copy(dst_hbm.at[wid], dst_v)
    pltpu.sync_copy(zeros_hbm.at[pl.ds(s * RPS, RPS)], acc.at[pl.ds(s * RPS, RPS)])
    plsc.subcore_barrier()

    pltpu.async_copy(g_hbm.at[src_v.at[0]], rows_a, sem_a)

    def step(i, carry):
        j = 2 * i
        pltpu.async_copy(g_hbm.at[src_v.at[j + 1]], rows_b, sem_b)
        pltpu.make_async_copy(g_hbm.at[src_v.at[j]], rows_a, sem_a).wait()
        pltpu.sync_copy(rows_a, acc.at[dst_v.at[j]], add=True)

        @pl.when(i < NCHUNK // 2 - 1)
        def _():
            pltpu.async_copy(g_hbm.at[src_v.at[j + 2]], rows_a, sem_a)

        pltpu.make_async_copy(g_hbm.at[src_v.at[j + 1]], rows_b, sem_b).wait()
        pltpu.sync_copy(rows_b, acc.at[dst_v.at[j + 1]], add=True)
        return carry

    lax.fori_loop(0, NCHUNK // 2, step, 0)
    plsc.subcore_barrier()
    pltpu.sync_copy(acc.at[pl.ds(s * RPS, RPS)],
                    out_hbm.at[pl.ds(c * NPAD + s * RPS, RPS)])


@functools.cache
def _sc_kernels():
    mesh = plsc.VectorSubcoreMesh(core_axis_name="c", subcore_axis_name="s")
    params = pltpu.CompilerParams(use_tc_tiling_on_sc=False)
    sc_deg = pl.kernel(
        _sc_deg_body,
        out_type=jax.ShapeDtypeStruct((NC * NPAD, WD), jnp.float32),
        mesh=mesh,
        compiler_params=params,
        scratch_types=[
            pltpu.VMEM((NCHUNK, K), jnp.int32),
            pltpu.VMEM((K, WD), jnp.float32),
            pltpu.VMEM_SHARED((NPAD, WD), jnp.float32),
        ],
    )
    sc_agg = pl.kernel(
        _sc_agg_body,
        out_type=jax.ShapeDtypeStruct((NC * NPAD, DP), jnp.float32),
        mesh=mesh,
        compiler_params=params,
        scratch_types=[
            pltpu.VMEM((NCHUNK, K), jnp.int32),
            pltpu.VMEM((NCHUNK, K), jnp.int32),
            pltpu.VMEM((K, DP), jnp.float32),
            pltpu.VMEM((K, DP), jnp.float32),
            pltpu.VMEM_SHARED((NPAD, DP), jnp.float32),
            pltpu.SemaphoreType.DMA,
            pltpu.SemaphoreType.DMA,
        ],
    )
    return sc_deg, sc_agg


_NB = 10
_BR = N // _NB


def _dinv_of(deg_ref):
    d = deg_ref[0, :, 0:1] + deg_ref[1, :, 0:1] + 1.0
    return lax.rsqrt(d)


def _tc1_body(deg_ref, x_ref, w_ref, o_ref):
    dinv = _dinv_of(deg_ref)
    h = jnp.dot(x_ref[...], w_ref[...], preferred_element_type=jnp.float32)
    o_ref[...] = h * dinv


def _tc2_body(deg_ref, a_ref, g_ref, w_ref, b_ref, o_ref):
    dinv = _dinv_of(deg_ref)
    z = jnp.maximum((a_ref[0] + a_ref[1] + g_ref[...]) * dinv + b_ref[...], 0.0)
    o_ref[...] = jnp.dot(z, w_ref[...], preferred_element_type=jnp.float32) * dinv


def _tc3_body(deg_ref, a_ref, g_ref, b_ref, o_ref):
    dinv = _dinv_of(deg_ref)
    o_ref[...] = (a_ref[0] + a_ref[1] + g_ref[...]) * dinv + b_ref[...]


_deg_spec = pl.BlockSpec((2, _BR, WD), lambda i: (0, i, 0))
_agg_spec = pl.BlockSpec((2, _BR, DP), lambda i: (0, i, 0))
_row_spec = pl.BlockSpec((_BR, DP), lambda i: (i, 0))
_out_spec = pl.BlockSpec((_BR, DP), lambda i: (i, 0))


def _tc1(degp, x, w1p):
    return pl.pallas_call(
        _tc1_body,
        grid=(_NB,),
        in_specs=[_deg_spec,
                  pl.BlockSpec((_BR, 128), lambda i: (i, 0)),
                  pl.BlockSpec((128, DP), lambda i: (0, 0))],
        out_specs=_out_spec,
        out_shape=jax.ShapeDtypeStruct((N, DP), jnp.float32),
    )(degp, x, w1p)


def _tc2(degp, aggp, g1, w2p, b1p):
    return pl.pallas_call(
        _tc2_body,
        grid=(_NB,),
        in_specs=[_deg_spec, _agg_spec, _row_spec,
                  pl.BlockSpec((DP, DP), lambda i: (0, 0)),
                  pl.BlockSpec((1, DP), lambda i: (0, 0))],
        out_specs=_out_spec,
        out_shape=jax.ShapeDtypeStruct((N, DP), jnp.float32),
    )(degp, aggp, g1, w2p, b1p)


def _tc3(degp, aggp, g2, b2p):
    return pl.pallas_call(
        _tc3_body,
        grid=(_NB,),
        in_specs=[_deg_spec, _agg_spec, _row_spec,
                  pl.BlockSpec((1, DP), lambda i: (0, 0))],
        out_specs=_out_spec,
        out_shape=jax.ShapeDtypeStruct((N, DP), jnp.float32),
    )(degp, aggp, g2, b2p)


def kernel(x, edge_index, W1, b1, W2, b2):
    ei = edge_index.astype(jnp.int32)
    src = jnp.concatenate([ei[0], jnp.zeros((EPAD - E,), jnp.int32)])
    dst = jnp.concatenate([ei[1], jnp.full((EPAD - E,), N, jnp.int32)])
    srcI = src.reshape(NW, NCHUNK, K)
    dstI = dst.reshape(NW, NCHUNK, K)

    w1p = jnp.zeros((128, DP), jnp.float32).at[:, :30].set(W1)
    b1p = jnp.zeros((1, DP), jnp.float32).at[0, :30].set(b1)
    w2p = jnp.zeros((DP, DP), jnp.float32).at[:30, :25].set(W2)
    b2p = jnp.zeros((1, DP), jnp.float32).at[0, :25].set(b2)

    z16 = jnp.zeros((NPAD, WD), jnp.float32)
    z32 = jnp.zeros((NPAD, DP), jnp.float32)
    ones16 = jnp.ones((K, WD), jnp.float32)

    sc_deg, sc_agg = _sc_kernels()
    degp = sc_deg(dstI, z16, ones16).reshape(NC, NPAD, WD)[:, :N, :]
    g1 = _tc1(degp, x, w1p)
    a1 = sc_agg(g1, srcI, dstI, z32).reshape(NC, NPAD, DP)[:, :N, :]
    g2 = _tc2(degp, a1, g1, w2p, b1p)
    a2 = sc_agg(g2, srcI, dstI, z32).reshape(NC, NPAD, DP)[:, :N, :]
    out = _tc3(degp, a2, g2, b2p)
    return out[:, :25]


# speedup vs baseline: 27.2735x; 1.0651x over previous
import functools

import jax
import jax.numpy as jnp
from jax import lax
from jax.experimental import pallas as pl
from jax.experimental.pallas import tpu as pltpu
from jax.experimental.pallas import tpu_sc as plsc

N = 10000
E = 320000
DP = 32
WD = 16
NC = 2
NS = 16
NW = NC * NS
K = 128
NCHUNK = 80
EPAD = NW * NCHUNK * K
NPAD = 12000
RPS = NPAD // NS

def _sc_deg_body(dst_hbm, zeros_hbm, ones_hbm, out_hbm, dst_v, ones_v, acc):
    c = lax.axis_index("c")
    s = lax.axis_index("s")
    wid = s * NC + c
    pltpu.sync_copy(dst_hbm.at[wid], dst_v)
    pltpu.sync_copy(ones_hbm, ones_v)
    pltpu.sync_copy(zeros_hbm.at[pl.ds(s * RPS, RPS)], acc.at[pl.ds(s * RPS, RPS)])
    plsc.subcore_barrier()

    def step(j, carry):
        pltpu.sync_copy(ones_v, acc.at[dst_v.at[j]], add=True)
        return carry

    lax.fori_loop(0, NCHUNK, step, 0)
    plsc.subcore_barrier()
    pltpu.sync_copy(acc.at[pl.ds(s * RPS, RPS)],
                    out_hbm.at[pl.ds(c * NPAD + s * RPS, RPS)])


def _sc_agg_body(g_hbm, src_hbm, dst_hbm, zeros_hbm, out_hbm,
                 src_v, dst_v, rows_a, rows_b, acc, sem_a, sem_b):
    c = lax.axis_index("c")
    s = lax.axis_index("s")
    wid = s * NC + c
    pltpu.sync_copy(src_hbm.at[wid], src_v)
    pltpu.sync_copy(dst_hbm.at[wid], dst_v)
    pltpu.sync_copy(zeros_hbm.at[pl.ds(s * RPS, RPS)], acc.at[pl.ds(s * RPS, RPS)])
    plsc.subcore_barrier()

    pltpu.async_copy(g_hbm.at[src_v.at[0]], rows_a, sem_a)

    def step(i, carry):
        j = 2 * i
        pltpu.async_copy(g_hbm.at[src_v.at[j + 1]], rows_b, sem_b)
        pltpu.make_async_copy(g_hbm.at[src_v.at[j]], rows_a, sem_a).wait()
        pltpu.sync_copy(rows_a, acc.at[dst_v.at[j]], add=True)

        @pl.when(i < NCHUNK // 2 - 1)
        def _():
            pltpu.async_copy(g_hbm.at[src_v.at[j + 2]], rows_a, sem_a)

        pltpu.make_async_copy(g_hbm.at[src_v.at[j + 1]], rows_b, sem_b).wait()
        pltpu.sync_copy(rows_b, acc.at[dst_v.at[j + 1]], add=True)
        return carry

    lax.fori_loop(0, NCHUNK // 2, step, 0)
    plsc.subcore_barrier()
    pltpu.sync_copy(acc.at[pl.ds(s * RPS, RPS)],
                    out_hbm.at[pl.ds(c * NPAD + s * RPS, RPS)])


@functools.cache
def _sc_kernels():
    mesh = plsc.VectorSubcoreMesh(core_axis_name="c", subcore_axis_name="s")
    params = pltpu.CompilerParams(use_tc_tiling_on_sc=False)
    sc_deg = pl.kernel(
        _sc_deg_body,
        out_type=jax.ShapeDtypeStruct((NC * NPAD, WD), jnp.float32),
        mesh=mesh,
        compiler_params=params,
        scratch_types=[
            pltpu.VMEM((NCHUNK, K), jnp.int32),
            pltpu.VMEM((K, WD), jnp.float32),
            pltpu.VMEM_SHARED((NPAD, WD), jnp.float32),
        ],
    )
    sc_agg = pl.kernel(
        _sc_agg_body,
        out_type=jax.ShapeDtypeStruct((NC * NPAD, DP), jnp.float32),
        mesh=mesh,
        compiler_params=params,
        scratch_types=[
            pltpu.VMEM((NCHUNK, K), jnp.int32),
            pltpu.VMEM((NCHUNK, K), jnp.int32),
            pltpu.VMEM((K, DP), jnp.float32),
            pltpu.VMEM((K, DP), jnp.float32),
            pltpu.VMEM_SHARED((NPAD, DP), jnp.float32),
            pltpu.SemaphoreType.DMA,
            pltpu.SemaphoreType.DMA,
        ],
    )
    return sc_deg, sc_agg


_NB = 5
_BR = N // _NB
_PB = NPAD // _BR


def _dinv_of(d0_ref, d1_ref):
    d = d0_ref[:, 0:1] + d1_ref[:, 0:1] + 1.0
    return lax.rsqrt(d)


def _tc1_body(d0_ref, d1_ref, x_ref, w_ref, o_ref):
    dinv = _dinv_of(d0_ref, d1_ref)
    h = jnp.dot(x_ref[...], w_ref[...], preferred_element_type=jnp.float32)
    o_ref[...] = h * dinv


def _tc2_body(d0_ref, d1_ref, a0_ref, a1_ref, g_ref, w_ref, b_ref, o_ref):
    dinv = _dinv_of(d0_ref, d1_ref)
    z = jnp.maximum((a0_ref[...] + a1_ref[...] + g_ref[...]) * dinv + b_ref[...],
                    0.0)
    o_ref[...] = jnp.dot(z, w_ref[...], preferred_element_type=jnp.float32) * dinv


def _tc3_body(d0_ref, d1_ref, a0_ref, a1_ref, g_ref, b_ref, o_ref):
    dinv = _dinv_of(d0_ref, d1_ref)
    o_ref[...] = (a0_ref[...] + a1_ref[...] + g_ref[...]) * dinv + b_ref[...]


_deg0_spec = pl.BlockSpec((_BR, WD), lambda i: (i, 0))
_deg1_spec = pl.BlockSpec((_BR, WD), lambda i: (_PB + i, 0))
_agg0_spec = pl.BlockSpec((_BR, DP), lambda i: (i, 0))
_agg1_spec = pl.BlockSpec((_BR, DP), lambda i: (_PB + i, 0))
_row_spec = pl.BlockSpec((_BR, DP), lambda i: (i, 0))


def _tc1(degp, x, w1p):
    return pl.pallas_call(
        _tc1_body,
        grid=(_NB,),
        in_specs=[_deg0_spec, _deg1_spec,
                  pl.BlockSpec((_BR, 128), lambda i: (i, 0)),
                  pl.BlockSpec((128, DP), lambda i: (0, 0))],
        out_specs=_row_spec,
        out_shape=jax.ShapeDtypeStruct((N, DP), jnp.float32),
    )(degp, degp, x, w1p)


def _tc2(degp, aggp, g1, w2p, b1p):
    return pl.pallas_call(
        _tc2_body,
        grid=(_NB,),
        in_specs=[_deg0_spec, _deg1_spec, _agg0_spec, _agg1_spec, _row_spec,
                  pl.BlockSpec((DP, DP), lambda i: (0, 0)),
                  pl.BlockSpec((1, DP), lambda i: (0, 0))],
        out_specs=_row_spec,
        out_shape=jax.ShapeDtypeStruct((N, DP), jnp.float32),
    )(degp, degp, aggp, aggp, g1, w2p, b1p)


def _tc3(degp, aggp, g2, b2p):
    return pl.pallas_call(
        _tc3_body,
        grid=(_NB,),
        in_specs=[_deg0_spec, _deg1_spec, _agg0_spec, _agg1_spec, _row_spec,
                  pl.BlockSpec((1, DP), lambda i: (0, 0))],
        out_specs=_row_spec,
        out_shape=jax.ShapeDtypeStruct((N, DP), jnp.float32),
    )(degp, degp, aggp, aggp, g2, b2p)


def kernel(x, edge_index, W1, b1, W2, b2):
    ei = edge_index.astype(jnp.int32)
    src = jnp.concatenate([ei[0], jnp.zeros((EPAD - E,), jnp.int32)])
    dst = jnp.concatenate([ei[1], jnp.full((EPAD - E,), N, jnp.int32)])
    srcI = src.reshape(NW, NCHUNK, K)
    dstI = dst.reshape(NW, NCHUNK, K)

    w1p = jnp.zeros((128, DP), jnp.float32).at[:, :30].set(W1)
    b1p = jnp.zeros((1, DP), jnp.float32).at[0, :30].set(b1)
    w2p = jnp.zeros((DP, DP), jnp.float32).at[:30, :25].set(W2)
    b2p = jnp.zeros((1, DP), jnp.float32).at[0, :25].set(b2)

    z16 = jnp.zeros((NPAD, WD), jnp.float32)
    z32 = jnp.zeros((NPAD, DP), jnp.float32)
    ones16 = jnp.ones((K, WD), jnp.float32)

    sc_deg, sc_agg = _sc_kernels()
    degp = sc_deg(dstI, z16, ones16)
    g1 = _tc1(degp, x, w1p)
    a1 = sc_agg(g1, srcI, dstI, z32)
    g2 = _tc2(degp, a1, g1, w2p, b1p)
    a2 = sc_agg(g2, srcI, dstI, z32)
    out = _tc3(degp, a2, g2, b2p)
    return out[:, :25]
